# baseline (speedup 1.0000x reference)
import functools

import jax
import jax.numpy as jnp
from jax import lax
from jax.experimental import pallas as pl
from jax.experimental.pallas import tpu as pltpu
from jax.experimental.pallas import tpu_sc as plsc

N = 10000
D = 128
NEG = 0.2
NC, NS, LANES = 2, 16, 16
NW = NC * NS
CPW = D // NW

C1 = 512
C2 = 8192
EWMAX = 10752
UNROLL = 16
PSHIFT = 14
PMASK = (1 << PSHIFT) - 1
EALIGN = NW * C1

RROWS = 264

_mesh_cache = []


def _get_mesh():
    if not _mesh_cache:
        _mesh_cache.append(plsc.VectorSubcoreMesh(
            core_axis_name="c", subcore_axis_name="s",
            num_cores=NC, num_subcores=NS))
    return _mesh_cache[0]


def _lrelu(x):
    return jnp.where(x > 0, x, NEG * x)


def _dense_body(m_ref, x_ref, o_ref):
    o_ref[...] = jnp.dot(m_ref[...], x_ref[...],
                         preferred_element_type=jnp.float32)


def _dense_proj(MT, xT):
    return pl.pallas_call(
        _dense_body,
        out_shape=jax.ShapeDtypeStruct((RROWS, N), jnp.float32),
    )(MT, xT)


def _sc1_body(rels, *refs):
    (hd, ht) = refs[0:2]
    nrel = len(rels)
    pks = refs[2:2 + nrel]
    exs = refs[2 + nrel:2 + 2 * nrel]
    sparts = refs[2 + 2 * nrel:2 + 3 * nrel]
    als_b, ald_b, sloc, pkb, exb = refs[2 + 3 * nrel:2 + 3 * nrel + 5]

    cid = lax.axis_index("c")
    sid = lax.axis_index("s")
    wid = sid * NC + cid

    iota = lax.iota(jnp.int32, LANES)
    zeros = jnp.zeros((LANES,), jnp.float32)

    for r, (h_is_d_src, h_is_d_dst, als_row, ald_row, e_real, e_pad) in \
            enumerate(rels):
        Hs = hd if h_is_d_src else ht
        Hd_ = hd if h_is_d_dst else ht
        pltpu.sync_copy(Hs.at[als_row], als_b)
        pltpu.sync_copy(Hd_.at[ald_row], ald_b)

        def _mx(i, m):
            return jnp.maximum(m, als_b[pl.ds(i * LANES, LANES)])
        mv = lax.fori_loop(0, N // LANES, _mx,
                           jnp.full((LANES,), -jnp.inf, jnp.float32))
        A = mv[0]
        for j in range(1, LANES):
            A = jnp.maximum(A, mv[j])

        def _z(i, _):
            sloc[pl.ds(i * LANES, LANES)] = zeros
            return 0
        lax.fori_loop(0, N // LANES, _z, 0)

        ew = e_pad // NW
        base0 = wid * ew
        pltpu.sync_copy(pks[r].at[pl.ds(base0, ew)], pkb.at[pl.ds(0, ew)])

        @plsc.parallel_loop(0, ew // LANES, unroll=8)
        def _vec(i):
            p16 = pkb[pl.ds(i * LANES, LANES)]
            s16 = jnp.bitwise_and(p16, PMASK)
            d16 = jnp.right_shift(p16, PSHIFT)
            a_s = plsc.load_gather(als_b, [s16])
            a_d = plsc.load_gather(ald_b, [d16])
            e = _lrelu(a_s + a_d)
            sh = _lrelu(a_d + A)
            eid = base0 + i * LANES + iota
            ex = jnp.where(eid < e_real, jnp.exp(e - sh), 0.0)
            exb[pl.ds(i * LANES, LANES)] = ex
            plsc.addupdate_scatter(sloc, [d16], ex)
        pltpu.sync_copy(exb.at[pl.ds(0, ew)], exs[r].at[pl.ds(base0, ew)])
        pltpu.sync_copy(sloc, sparts[r].at[wid])


def _run_sc1(rels, HD, HT, edge_args):
    nrel = len(rels)
    out_type = (
        [jax.ShapeDtypeStruct((r[5],), jnp.float32) for r in rels]
        + [jax.ShapeDtypeStruct((NW, N), jnp.float32) for _ in rels])
    scratch = [
        pltpu.VMEM((N,), jnp.float32),
        pltpu.VMEM((N,), jnp.float32),
        pltpu.VMEM((N,), jnp.float32),
        pltpu.VMEM((EWMAX,), jnp.int32),
        pltpu.VMEM((EWMAX,), jnp.float32),
    ]
    fn = pl.kernel(
        functools.partial(_sc1_body, rels),
        out_type=out_type,
        mesh=_get_mesh(),
        scratch_types=scratch,
        compiler_params=pltpu.CompilerParams(needs_layout_passes=False),
    )
    res = fn(HD, HT, *edge_args)
    return res[:nrel], res[nrel:]


def _sc2_body(rels, *refs):
    (hd, ht) = refs[0:2]
    nrel = len(rels)
    pks = refs[2:2 + 2 * nrel:2]
    exs = refs[3:2 + 2 * nrel:2]
    numers = refs[2 + 2 * nrel:2 + 3 * nrel]
    (hloc, nloc, pkb0, exb0, pkb1, exb1,
     sem0, sem1) = refs[2 + 3 * nrel:2 + 3 * nrel + 8]
    bufs = ((pkb0, exb0, sem0), (pkb1, exb1, sem1))

    cid = lax.axis_index("c")
    sid = lax.axis_index("s")
    wid = sid * NC + cid
    row0 = wid * CPW

    zeros = jnp.zeros((LANES,), jnp.float32)
    jvecs = [jnp.full((LANES,), j, jnp.int32) for j in range(CPW)]

    for r, (h_is_d_src, hoff, e_pad) in enumerate(rels):
        H = hd if h_is_d_src else ht
        pltpu.sync_copy(H.at[pl.ds(hoff + row0, CPW)], hloc)

        def _z(i, _):
            for j in range(CPW):
                nloc[j, pl.ds(i * LANES, LANES)] = zeros
            return 0
        lax.fori_loop(0, N // LANES, _z, 0)

        def _issue(k, b):
            pb, eb, sem = bufs[b]
            base = k * C2
            pltpu.async_copy(pks[r].at[pl.ds(base, C2)], pb, sem)
            pltpu.async_copy(exs[r].at[pl.ds(base, C2)], eb, sem)

        def _wait(b):
            pb, eb, sem = bufs[b]
            pltpu.make_async_copy(pks[r].at[pl.ds(0, C2)], pb, sem).wait()
            pltpu.make_async_copy(exs[r].at[pl.ds(0, C2)], eb, sem).wait()

        nch = e_pad // C2
        _issue(0, 0)
        _issue(1, 1)

        def _chunk2(kk, _):
            for b in (0, 1):
                k = 2 * kk + b
                _wait(b)
                pb, eb, sem = bufs[b]

                @plsc.parallel_loop(0, C2 // LANES, unroll=UNROLL)
                def _vec(ii):
                    off = ii * LANES
                    p16 = pb[pl.ds(off, LANES)]
                    s16 = jnp.bitwise_and(p16, PMASK)
                    d16 = jnp.right_shift(p16, PSHIFT)
                    ex = eb[pl.ds(off, LANES)]
                    for j in range(CPW):
                        h = plsc.load_gather(hloc, [jvecs[j], s16])
                        plsc.addupdate_scatter(
                            nloc, [jvecs[j], d16], h * ex)

                @pl.when(k + 2 < nch)
                def _():
                    _issue(k + 2, b)
            return 0
        lax.fori_loop(0, nch // 2, _chunk2, 0)
        pltpu.sync_copy(nloc, numers[r].at[pl.ds(row0, CPW)])


def _run_sc2(rels, HD, HT, edge_args):
    nrel = len(rels)
    out_type = [jax.ShapeDtypeStruct((D, N), jnp.float32) for _ in rels]
    scratch = [
        pltpu.VMEM((CPW, N), jnp.float32),
        pltpu.VMEM((CPW, N), jnp.float32),
        pltpu.VMEM((C2,), jnp.int32),
        pltpu.VMEM((C2,), jnp.float32),
        pltpu.VMEM((C2,), jnp.int32),
        pltpu.VMEM((C2,), jnp.float32),
        pltpu.SemaphoreType.DMA,
        pltpu.SemaphoreType.DMA,
    ]
    fn = pl.kernel(
        functools.partial(_sc2_body, rels),
        out_type=out_type,
        mesh=_get_mesh(),
        scratch_types=scratch,
        compiler_params=pltpu.CompilerParams(needs_layout_passes=False),
    )
    return fn(HD, HT, *edge_args)


def _fin_body(n0_ref, s0_ref, b0_ref, n1_ref, s1_ref, b1_ref, o_ref):
    s0 = jnp.sum(s0_ref[...], axis=0, keepdims=True) + 1e-16
    s1 = jnp.sum(s1_ref[...], axis=0, keepdims=True) + 1e-16
    o = n0_ref[...] / s0 + b0_ref[...] + n1_ref[...] / s1 + b1_ref[...]
    o_ref[...] = jnp.maximum(o, 0.0)


def _finalize(numer0, spart0, b0, numer1, spart1, b1):
    return pl.pallas_call(
        _fin_body,
        out_shape=jax.ShapeDtypeStruct((D, N), jnp.float32),
    )(numer0, spart0, b0[:, None], numer1, spart1, b1[:, None])


def _pad_edges(src, dst, add_loops):
    if add_loops:
        loop = jnp.arange(N, dtype=src.dtype)
        src = jnp.concatenate([src, loop])
        dst = jnp.concatenate([dst, loop])
    e = src.shape[0]
    e_pad = -(-e // EALIGN) * EALIGN
    pad = e_pad - e
    packed = src + dst * (PMASK + 1)
    packed = jnp.concatenate([packed, jnp.zeros((pad,), packed.dtype)])
    return packed, e, e_pad


def _build_MT(W_a, W_b, as_a, as_b, ad_a, ad_x, W_x):
    cols = jnp.stack([W_a @ as_a, W_b @ as_b, W_a @ ad_a, W_x @ ad_x], axis=1)
    M = jnp.concatenate([W_a, W_b, cols, jnp.zeros((D, 4), jnp.float32)],
                        axis=1)
    return M.T


def _layer(xdT, xtT, W, ass, ad, b, ei):
    MTd = _build_MT(W["dd"], W["db"], ass["dd"], ass["db"], ad["dd"],
                    ad["td"], W["td"])
    MTt = _build_MT(W["tt"], W["td"], ass["tt"], ass["td"], ad["tt"],
                    ad["db"], W["db"])
    HD = _dense_proj(MTd, xdT)
    HT = _dense_proj(MTt, xtT)

    meta = {
        "dd": (True, True, 2 * D, 2 * D + 2, 0),
        "td": (False, True, 2 * D + 1, 2 * D + 3, D),
        "db": (True, False, 2 * D + 1, 2 * D + 3, D),
        "tt": (False, False, 2 * D, 2 * D + 2, 0),
    }
    order = ["dd", "td", "db", "tt"]

    rels1, args1, rels2 = [], [], []
    for t in order:
        sd, dd_, alsr, aldr, hoff = meta[t]
        packed, e_real, e_pad = ei[t]
        rels1.append((sd, dd_, alsr, aldr, e_real, e_pad))
        args1 += [packed]
        rels2.append((sd, hoff, e_pad))
    exs, sparts = _run_sc1(rels1, HD, HT, args1)

    args2 = []
    for i, t in enumerate(order):
        packed, _, _ = ei[t]
        args2 += [packed, exs[i]]
    numers = _run_sc2(rels2, HD, HT, args2)

    nm = dict(zip(order, numers))
    sp = dict(zip(order, sparts))
    outdT = _finalize(nm["dd"], sp["dd"], b["dd"], nm["td"], sp["td"], b["td"])
    outtT = _finalize(nm["db"], sp["db"], b["db"], nm["tt"], sp["tt"], b["tt"])
    return outdT, outtT


def kernel(x_drug, x_target, ei_dd, ei_db, ei_td, ei_tt,
           W1dd, as1dd, ad1dd, b1dd,
           W1db, as1db, ad1db, b1db,
           W1td, as1td, ad1td, b1td,
           W1tt, as1tt, ad1tt, b1tt,
           W2dd, as2dd, ad2dd, b2dd,
           W2db, as2db, ad2db, b2db,
           W2td, as2td, ad2td, b2td,
           W2tt, as2tt, ad2tt, b2tt):
    ei = {
        "dd": _pad_edges(ei_dd[0], ei_dd[1], True),
        "db": _pad_edges(ei_db[0], ei_db[1], False),
        "td": _pad_edges(ei_td[0], ei_td[1], False),
        "tt": _pad_edges(ei_tt[0], ei_tt[1], True),
    }
    p = dict(locals())
    xdT = x_drug.T
    xtT = x_target.T
    for L in ("1", "2"):
        W = {t: p["W" + L + t] for t in ("dd", "db", "td", "tt")}
        ass = {t: p["as" + L + t] for t in ("dd", "db", "td", "tt")}
        ad = {t: p["ad" + L + t] for t in ("dd", "db", "td", "tt")}
        b = {t: p["b" + L + t] for t in ("dd", "db", "td", "tt")}
        xdT, xtT = _layer(xdT, xtT, W, ass, ad, b, ei)
    return (xdT.T, xtT.T)


# speedup vs baseline: 1.0565x; 1.0565x over previous
import functools

import jax
import jax.numpy as jnp
from jax import lax
from jax.experimental import pallas as pl
from jax.experimental.pallas import tpu as pltpu
from jax.experimental.pallas import tpu_sc as plsc

N = 10000
D = 128
NEG = 0.2
NC, NS, LANES = 2, 16, 16
NW = NC * NS
CPW = D // NW

C1 = 512
C2 = 8192
EWMAX = 10752
UNROLL = 8
PSHIFT = 14
PMASK = (1 << PSHIFT) - 1
EALIGN = NW * C1

RROWS = 264

_mesh_cache = []


def _get_mesh():
    if not _mesh_cache:
        _mesh_cache.append(plsc.VectorSubcoreMesh(
            core_axis_name="c", subcore_axis_name="s",
            num_cores=NC, num_subcores=NS))
    return _mesh_cache[0]


def _lrelu(x):
    return jnp.where(x > 0, x, NEG * x)


def _dense_body(md_ref, xd_ref, mt_ref, xt_ref, od_ref, ot_ref):
    od_ref[...] = jnp.dot(md_ref[...], xd_ref[...],
                          preferred_element_type=jnp.float32)
    ot_ref[...] = jnp.dot(mt_ref[...], xt_ref[...],
                          preferred_element_type=jnp.float32)


def _dense_proj(MTd, xdT, MTt, xtT):
    return pl.pallas_call(
        _dense_body,
        out_shape=[jax.ShapeDtypeStruct((RROWS, N), jnp.float32)] * 2,
    )(MTd, xdT, MTt, xtT)


def _sc1_body(rels, *refs):
    (hd, ht) = refs[0:2]
    nrel = len(rels)
    pks = refs[2:2 + nrel]
    exs = refs[2 + nrel:2 + 2 * nrel]
    sparts = refs[2 + 2 * nrel:2 + 3 * nrel]
    als_b, ald_b, sloc, pkb, exb = refs[2 + 3 * nrel:2 + 3 * nrel + 5]

    cid = lax.axis_index("c")
    sid = lax.axis_index("s")
    wid = sid * NC + cid

    iota = lax.iota(jnp.int32, LANES)
    zeros = jnp.zeros((LANES,), jnp.float32)

    for r, (h_is_d_src, h_is_d_dst, als_row, ald_row, e_real, e_pad) in \
            enumerate(rels):
        Hs = hd if h_is_d_src else ht
        Hd_ = hd if h_is_d_dst else ht
        pltpu.sync_copy(Hs.at[als_row], als_b)
        pltpu.sync_copy(Hd_.at[ald_row], ald_b)

        def _mx(i, m):
            return jnp.maximum(m, als_b[pl.ds(i * LANES, LANES)])
        mv = lax.fori_loop(0, N // LANES, _mx,
                           jnp.full((LANES,), -jnp.inf, jnp.float32))
        A = mv[0]
        for j in range(1, LANES):
            A = jnp.maximum(A, mv[j])

        def _z(i, _):
            sloc[pl.ds(i * LANES, LANES)] = zeros
            return 0
        lax.fori_loop(0, N // LANES, _z, 0)

        ew = e_pad // NW
        base0 = wid * ew
        pltpu.sync_copy(pks[r].at[pl.ds(base0, ew)], pkb.at[pl.ds(0, ew)])

        @plsc.parallel_loop(0, ew // LANES, unroll=4)
        def _vec(i):
            p16 = pkb[pl.ds(i * LANES, LANES)]
            s16 = jnp.bitwise_and(p16, PMASK)
            d16 = jnp.right_shift(p16, PSHIFT)
            a_s = plsc.load_gather(als_b, [s16])
            a_d = plsc.load_gather(ald_b, [d16])
            e = _lrelu(a_s + a_d)
            sh = _lrelu(a_d + A)
            eid = base0 + i * LANES + iota
            ex = jnp.where(eid < e_real, jnp.exp(e - sh), 0.0)
            exb[pl.ds(i * LANES, LANES)] = ex
            plsc.addupdate_scatter(sloc, [d16], ex)
        pltpu.sync_copy(exb.at[pl.ds(0, ew)], exs[r].at[pl.ds(base0, ew)])
        pltpu.sync_copy(sloc, sparts[r].at[wid])


def _run_sc1(rels, HD, HT, edge_args):
    nrel = len(rels)
    out_type = (
        [jax.ShapeDtypeStruct((r[5],), jnp.float32) for r in rels]
        + [jax.ShapeDtypeStruct((NW, N), jnp.float32) for _ in rels])
    scratch = [
        pltpu.VMEM((N,), jnp.float32),
        pltpu.VMEM((N,), jnp.float32),
        pltpu.VMEM((N,), jnp.float32),
        pltpu.VMEM((EWMAX,), jnp.int32),
        pltpu.VMEM((EWMAX,), jnp.float32),
    ]
    fn = pl.kernel(
        functools.partial(_sc1_body, rels),
        out_type=out_type,
        mesh=_get_mesh(),
        scratch_types=scratch,
        compiler_params=pltpu.CompilerParams(needs_layout_passes=False),
    )
    res = fn(HD, HT, *edge_args)
    return res[:nrel], res[nrel:]


def _sc2_body(rels, *refs):
    (hd, ht) = refs[0:2]
    nrel = len(rels)
    pks = refs[2:2 + 2 * nrel:2]
    exs = refs[3:2 + 2 * nrel:2]
    numers = refs[2 + 2 * nrel:2 + 3 * nrel]
    (hloc, nloc, pkb0, exb0, pkb1, exb1,
     sem0, sem1) = refs[2 + 3 * nrel:2 + 3 * nrel + 8]
    bufs = ((pkb0, exb0, sem0), (pkb1, exb1, sem1))

    cid = lax.axis_index("c")
    sid = lax.axis_index("s")
    wid = sid * NC + cid
    row0 = wid * CPW

    zeros = jnp.zeros((LANES,), jnp.float32)
    jvecs = [jnp.full((LANES,), j, jnp.int32) for j in range(CPW)]

    for r, (h_is_d_src, hoff, e_pad) in enumerate(rels):
        H = hd if h_is_d_src else ht
        pltpu.sync_copy(H.at[pl.ds(hoff + row0, CPW)], hloc)

        def _z(i, _):
            for j in range(CPW):
                nloc[j, pl.ds(i * LANES, LANES)] = zeros
            return 0
        lax.fori_loop(0, N // LANES, _z, 0)

        def _issue(k, b):
            pb, eb, sem = bufs[b]
            base = k * C2
            pltpu.async_copy(pks[r].at[pl.ds(base, C2)], pb, sem)
            pltpu.async_copy(exs[r].at[pl.ds(base, C2)], eb, sem)

        def _wait(b):
            pb, eb, sem = bufs[b]
            pltpu.make_async_copy(pks[r].at[pl.ds(0, C2)], pb, sem).wait()
            pltpu.make_async_copy(exs[r].at[pl.ds(0, C2)], eb, sem).wait()

        nch = e_pad // C2
        _issue(0, 0)
        _issue(1, 1)

        def _chunk2(kk, _):
            for b in (0, 1):
                k = 2 * kk + b
                _wait(b)
                pb, eb, sem = bufs[b]

                @plsc.parallel_loop(0, C2 // LANES, unroll=UNROLL)
                def _vec(ii):
                    off = ii * LANES
                    p16 = pb[pl.ds(off, LANES)]
                    s16 = jnp.bitwise_and(p16, PMASK)
                    d16 = jnp.right_shift(p16, PSHIFT)
                    ex = eb[pl.ds(off, LANES)]
                    for j in range(CPW):
                        h = plsc.load_gather(hloc, [jvecs[j], s16])
                        plsc.addupdate_scatter(
                            nloc, [jvecs[j], d16], h * ex)

                @pl.when(k + 2 < nch)
                def _():
                    _issue(k + 2, b)
            return 0
        lax.fori_loop(0, nch // 2, _chunk2, 0)
        pltpu.sync_copy(nloc, numers[r].at[pl.ds(row0, CPW)])


def _run_sc2(rels, HD, HT, edge_args):
    nrel = len(rels)
    out_type = [jax.ShapeDtypeStruct((D, N), jnp.float32) for _ in rels]
    scratch = [
        pltpu.VMEM((CPW, N), jnp.float32),
        pltpu.VMEM((CPW, N), jnp.float32),
        pltpu.VMEM((C2,), jnp.int32),
        pltpu.VMEM((C2,), jnp.float32),
        pltpu.VMEM((C2,), jnp.int32),
        pltpu.VMEM((C2,), jnp.float32),
        pltpu.SemaphoreType.DMA,
        pltpu.SemaphoreType.DMA,
    ]
    fn = pl.kernel(
        functools.partial(_sc2_body, rels),
        out_type=out_type,
        mesh=_get_mesh(),
        scratch_types=scratch,
        compiler_params=pltpu.CompilerParams(needs_layout_passes=False),
    )
    return fn(HD, HT, *edge_args)


def _fin_body(*refs):
    (n0, s0, b0, n1, s1, b1, n2, s2, b2, n3, s3, b3, od, ot) = refs

    def one(na, sa, ba, nb, sb, bb):
        ssa = jnp.sum(sa[...], axis=0, keepdims=True) + 1e-16
        ssb = jnp.sum(sb[...], axis=0, keepdims=True) + 1e-16
        return jnp.maximum(na[...] / ssa + ba[...] + nb[...] / ssb + bb[...],
                           0.0)

    od[...] = one(n0, s0, b0, n1, s1, b1)
    ot[...] = one(n2, s2, b2, n3, s3, b3)


def _finalize(args_d, args_t):
    def flat(a):
        n0, s0, b0, n1, s1, b1 = a
        return [n0, s0, b0[:, None], n1, s1, b1[:, None]]
    return pl.pallas_call(
        _fin_body,
        out_shape=[jax.ShapeDtypeStruct((D, N), jnp.float32)] * 2,
    )(*flat(args_d), *flat(args_t))


def _pad_edges(src, dst, add_loops):
    if add_loops:
        loop = jnp.arange(N, dtype=src.dtype)
        src = jnp.concatenate([src, loop])
        dst = jnp.concatenate([dst, loop])
    e = src.shape[0]
    e_pad = -(-e // EALIGN) * EALIGN
    pad = e_pad - e
    packed = src + dst * (PMASK + 1)
    packed = jnp.concatenate([packed, jnp.zeros((pad,), packed.dtype)])
    return packed, e, e_pad


def _build_MT(W_a, W_b, as_a, as_b, ad_a, ad_x, W_x):
    cols = jnp.stack([W_a @ as_a, W_b @ as_b, W_a @ ad_a, W_x @ ad_x], axis=1)
    M = jnp.concatenate([W_a, W_b, cols, jnp.zeros((D, 4), jnp.float32)],
                        axis=1)
    return M.T


def _layer(xdT, xtT, W, ass, ad, b, ei):
    MTd = _build_MT(W["dd"], W["db"], ass["dd"], ass["db"], ad["dd"],
                    ad["td"], W["td"])
    MTt = _build_MT(W["tt"], W["td"], ass["tt"], ass["td"], ad["tt"],
                    ad["db"], W["db"])
    HD, HT = _dense_proj(MTd, xdT, MTt, xtT)

    meta = {
        "dd": (True, True, 2 * D, 2 * D + 2, 0),
        "td": (False, True, 2 * D + 1, 2 * D + 3, D),
        "db": (True, False, 2 * D + 1, 2 * D + 3, D),
        "tt": (False, False, 2 * D, 2 * D + 2, 0),
    }
    order = ["dd", "td", "db", "tt"]

    rels1, args1, rels2 = [], [], []
    for t in order:
        sd, dd_, alsr, aldr, hoff = meta[t]
        packed, e_real, e_pad = ei[t]
        rels1.append((sd, dd_, alsr, aldr, e_real, e_pad))
        args1 += [packed]
        rels2.append((sd, hoff, e_pad))
    exs, sparts = _run_sc1(rels1, HD, HT, args1)

    args2 = []
    for i, t in enumerate(order):
        packed, _, _ = ei[t]
        args2 += [packed, exs[i]]
    numers = _run_sc2(rels2, HD, HT, args2)

    nm = dict(zip(order, numers))
    sp = dict(zip(order, sparts))
    outdT, outtT = _finalize(
        (nm["dd"], sp["dd"], b["dd"], nm["td"], sp["td"], b["td"]),
        (nm["db"], sp["db"], b["db"], nm["tt"], sp["tt"], b["tt"]))
    return outdT, outtT


def kernel(x_drug, x_target, ei_dd, ei_db, ei_td, ei_tt,
           W1dd, as1dd, ad1dd, b1dd,
           W1db, as1db, ad1db, b1db,
           W1td, as1td, ad1td, b1td,
           W1tt, as1tt, ad1tt, b1tt,
           W2dd, as2dd, ad2dd, b2dd,
           W2db, as2db, ad2db, b2db,
           W2td, as2td, ad2td, b2td,
           W2tt, as2tt, ad2tt, b2tt):
    ei = {
        "dd": _pad_edges(ei_dd[0], ei_dd[1], True),
        "db": _pad_edges(ei_db[0], ei_db[1], False),
        "td": _pad_edges(ei_td[0], ei_td[1], False),
        "tt": _pad_edges(ei_tt[0], ei_tt[1], True),
    }
    p = dict(locals())
    xdT = x_drug.T
    xtT = x_target.T
    for L in ("1", "2"):
        W = {t: p["W" + L + t] for t in ("dd", "db", "td", "tt")}
        ass = {t: p["as" + L + t] for t in ("dd", "db", "td", "tt")}
        ad = {t: p["ad" + L + t] for t in ("dd", "db", "td", "tt")}
        b = {t: p["b" + L + t] for t in ("dd", "db", "td", "tt")}
        xdT, xtT = _layer(xdT, xtT, W, ass, ad, b, ei)
    return (xdT.T, xtT.T)


# speedup vs baseline: 1.1417x; 1.0807x over previous
import functools

import jax
import jax.numpy as jnp
from jax import lax
from jax.experimental import pallas as pl
from jax.experimental.pallas import tpu as pltpu
from jax.experimental.pallas import tpu_sc as plsc

N = 10000
D = 128
NEG = 0.2
NC, NS, LANES = 2, 16, 16
NW = NC * NS
CPW = D // NW

C1 = 512
C2 = 8192
EWMAX = 10496
UNROLL = 8
PSHIFT = 14
PMASK = (1 << PSHIFT) - 1
EALIGN = 8192

RROWS = 264

_mesh_cache = []


def _get_mesh():
    if not _mesh_cache:
        _mesh_cache.append(plsc.VectorSubcoreMesh(
            core_axis_name="c", subcore_axis_name="s",
            num_cores=NC, num_subcores=NS))
    return _mesh_cache[0]


def _lrelu(x):
    return jnp.where(x > 0, x, NEG * x)


def _dense_body(md_ref, xd_ref, mt_ref, xt_ref, od_ref, ot_ref):
    od_ref[...] = jnp.dot(md_ref[...], xd_ref[...],
                          preferred_element_type=jnp.float32)
    ot_ref[...] = jnp.dot(mt_ref[...], xt_ref[...],
                          preferred_element_type=jnp.float32)


def _dense_proj(MTd, xdT, MTt, xtT):
    return pl.pallas_call(
        _dense_body,
        out_shape=[jax.ShapeDtypeStruct((RROWS, N), jnp.float32)] * 2,
    )(MTd, xdT, MTt, xtT)


def _sc1_body(rels, *refs):
    (hd, ht) = refs[0:2]
    nrel = len(rels)
    pks = refs[2:2 + nrel]
    exs = refs[2 + nrel:2 + 2 * nrel]
    sparts = refs[2 + 2 * nrel:2 + 3 * nrel]
    als_b, ald_b, sloc, pkb, exb = refs[2 + 3 * nrel:2 + 3 * nrel + 5]

    cid = lax.axis_index("c")
    sid = lax.axis_index("s")
    wid = sid * NC + cid

    iota = lax.iota(jnp.int32, LANES)
    zeros = jnp.zeros((LANES,), jnp.float32)

    for r, (h_is_d_src, h_is_d_dst, als_row, ald_row, e_real, e_pad) in \
            enumerate(rels):
        Hs = hd if h_is_d_src else ht
        Hd_ = hd if h_is_d_dst else ht
        pltpu.sync_copy(Hs.at[als_row], als_b)
        pltpu.sync_copy(Hd_.at[ald_row], ald_b)

        def _mx(i, m):
            return jnp.maximum(m, als_b[pl.ds(i * LANES, LANES)])
        mv = lax.fori_loop(0, N // LANES, _mx,
                           jnp.full((LANES,), -jnp.inf, jnp.float32))
        A = mv[0]
        for j in range(1, LANES):
            A = jnp.maximum(A, mv[j])

        def _z(i, _):
            sloc[pl.ds(i * LANES, LANES)] = zeros
            return 0
        lax.fori_loop(0, N // LANES, _z, 0)

        ew = e_pad // NW
        base0 = wid * ew
        pltpu.sync_copy(pks[r].at[pl.ds(base0, ew)], pkb.at[pl.ds(0, ew)])

        @plsc.parallel_loop(0, ew // LANES, unroll=8)
        def _vec(i):
            p16 = pkb[pl.ds(i * LANES, LANES)]
            s16 = jnp.bitwise_and(p16, PMASK)
            d16 = jnp.right_shift(p16, PSHIFT)
            a_s = plsc.load_gather(als_b, [s16])
            a_d = plsc.load_gather(ald_b, [d16])
            e = _lrelu(a_s + a_d)
            sh = _lrelu(a_d + A)
            eid = base0 + i * LANES + iota
            ex = jnp.where(eid < e_real, jnp.exp(e - sh), 0.0)
            exb[pl.ds(i * LANES, LANES)] = ex
            plsc.addupdate_scatter(sloc, [d16], ex)
        pltpu.sync_copy(exb.at[pl.ds(0, ew)], exs[r].at[pl.ds(base0, ew)])
        pltpu.sync_copy(sloc, sparts[r].at[wid])


def _run_sc1(rels, HD, HT, edge_args):
    nrel = len(rels)
    out_type = (
        [jax.ShapeDtypeStruct((r[5],), jnp.float32) for r in rels]
        + [jax.ShapeDtypeStruct((NW, N), jnp.float32) for _ in rels])
    scratch = [
        pltpu.VMEM((N,), jnp.float32),
        pltpu.VMEM((N,), jnp.float32),
        pltpu.VMEM((N,), jnp.float32),
        pltpu.VMEM((EWMAX,), jnp.int32),
        pltpu.VMEM((EWMAX,), jnp.float32),
    ]
    fn = pl.kernel(
        functools.partial(_sc1_body, rels),
        out_type=out_type,
        mesh=_get_mesh(),
        scratch_types=scratch,
        compiler_params=pltpu.CompilerParams(needs_layout_passes=False),
    )
    res = fn(HD, HT, *edge_args)
    return res[:nrel], res[nrel:]


def _sc2_body(rels, *refs):
    (hd, ht) = refs[0:2]
    nrel = len(rels)
    pks = refs[2:2 + 2 * nrel:2]
    exs = refs[3:2 + 2 * nrel:2]
    numers = refs[2 + 2 * nrel:2 + 3 * nrel]
    (hloc, nloc, pkb0, exb0, pkb1, exb1,
     sem0, sem1) = refs[2 + 3 * nrel:2 + 3 * nrel + 8]
    bufs = ((pkb0, exb0, sem0), (pkb1, exb1, sem1))

    cid = lax.axis_index("c")
    sid = lax.axis_index("s")
    wid = sid * NC + cid
    row0 = wid * CPW

    zeros = jnp.zeros((LANES,), jnp.float32)
    jvecs = [jnp.full((LANES,), j, jnp.int32) for j in range(CPW)]

    for r, (h_is_d_src, hoff, e_pad) in enumerate(rels):
        H = hd if h_is_d_src else ht
        pltpu.sync_copy(H.at[pl.ds(hoff + row0, CPW)], hloc)

        def _z(i, _):
            for j in range(CPW):
                nloc[j, pl.ds(i * LANES, LANES)] = zeros
            return 0
        lax.fori_loop(0, N // LANES, _z, 0)

        def _issue(k, b):
            pb, eb, sem = bufs[b]
            base = k * C2
            pltpu.async_copy(pks[r].at[pl.ds(base, C2)], pb, sem)
            pltpu.async_copy(exs[r].at[pl.ds(base, C2)], eb, sem)

        def _wait(b):
            pb, eb, sem = bufs[b]
            pltpu.make_async_copy(pks[r].at[pl.ds(0, C2)], pb, sem).wait()
            pltpu.make_async_copy(exs[r].at[pl.ds(0, C2)], eb, sem).wait()

        def _proc(b):
            pb, eb, sem = bufs[b]

            @plsc.parallel_loop(0, C2 // LANES, unroll=UNROLL)
            def _vec(ii):
                off = ii * LANES
                p16 = pb[pl.ds(off, LANES)]
                s16 = jnp.bitwise_and(p16, PMASK)
                d16 = jnp.right_shift(p16, PSHIFT)
                ex = eb[pl.ds(off, LANES)]
                for j in range(CPW):
                    h = plsc.load_gather(hloc, [jvecs[j], s16])
                    plsc.addupdate_scatter(
                        nloc, [jvecs[j], d16], h * ex)

        nch = e_pad // C2
        _issue(0, 0)
        _issue(1, 1)

        def _chunk2(kk, _):
            for b in (0, 1):
                k = 2 * kk + b
                _wait(b)
                _proc(b)

                @pl.when(k + 2 < nch)
                def _():
                    _issue(k + 2, b)
            return 0
        lax.fori_loop(0, nch // 2, _chunk2, 0)
        if nch % 2:
            bl = (nch - 1) % 2
            _wait(bl)
            _proc(bl)
        pltpu.sync_copy(nloc, numers[r].at[pl.ds(row0, CPW)])


def _run_sc2(rels, HD, HT, edge_args):
    nrel = len(rels)
    out_type = [jax.ShapeDtypeStruct((D, N), jnp.float32) for _ in rels]
    scratch = [
        pltpu.VMEM((CPW, N), jnp.float32),
        pltpu.VMEM((CPW, N), jnp.float32),
        pltpu.VMEM((C2,), jnp.int32),
        pltpu.VMEM((C2,), jnp.float32),
        pltpu.VMEM((C2,), jnp.int32),
        pltpu.VMEM((C2,), jnp.float32),
        pltpu.SemaphoreType.DMA,
        pltpu.SemaphoreType.DMA,
    ]
    fn = pl.kernel(
        functools.partial(_sc2_body, rels),
        out_type=out_type,
        mesh=_get_mesh(),
        scratch_types=scratch,
        compiler_params=pltpu.CompilerParams(needs_layout_passes=False),
    )
    return fn(HD, HT, *edge_args)


def _fin_body(*refs):
    (n0, s0, b0, n1, s1, b1, n2, s2, b2, n3, s3, b3, od, ot) = refs

    def one(na, sa, ba, nb, sb, bb):
        ssa = jnp.sum(sa[...], axis=0, keepdims=True) + 1e-16
        ssb = jnp.sum(sb[...], axis=0, keepdims=True) + 1e-16
        return jnp.maximum(na[...] / ssa + ba[...] + nb[...] / ssb + bb[...],
                           0.0)

    od[...] = one(n0, s0, b0, n1, s1, b1)
    ot[...] = one(n2, s2, b2, n3, s3, b3)


def _finalize(args_d, args_t):
    def flat(a):
        n0, s0, b0, n1, s1, b1 = a
        return [n0, s0, b0[:, None], n1, s1, b1[:, None]]
    return pl.pallas_call(
        _fin_body,
        out_shape=[jax.ShapeDtypeStruct((D, N), jnp.float32)] * 2,
    )(*flat(args_d), *flat(args_t))


def _pad_edges(src, dst, add_loops):
    if add_loops:
        loop = jnp.arange(N, dtype=src.dtype)
        src = jnp.concatenate([src, loop])
        dst = jnp.concatenate([dst, loop])
    e = src.shape[0]
    e_pad = -(-e // EALIGN) * EALIGN
    pad = e_pad - e
    packed = src + dst * (PMASK + 1)
    packed = jnp.concatenate([packed, jnp.zeros((pad,), packed.dtype)])
    return packed, e, e_pad


def _build_MT(W_a, W_b, as_a, as_b, ad_a, ad_x, W_x):
    cols = jnp.stack([W_a @ as_a, W_b @ as_b, W_a @ ad_a, W_x @ ad_x], axis=1)
    M = jnp.concatenate([W_a, W_b, cols, jnp.zeros((D, 4), jnp.float32)],
                        axis=1)
    return M.T


def _layer(xdT, xtT, W, ass, ad, b, ei):
    MTd = _build_MT(W["dd"], W["db"], ass["dd"], ass["db"], ad["dd"],
                    ad["td"], W["td"])
    MTt = _build_MT(W["tt"], W["td"], ass["tt"], ass["td"], ad["tt"],
                    ad["db"], W["db"])
    HD, HT = _dense_proj(MTd, xdT, MTt, xtT)

    meta = {
        "dd": (True, True, 2 * D, 2 * D + 2, 0),
        "td": (False, True, 2 * D + 1, 2 * D + 3, D),
        "db": (True, False, 2 * D + 1, 2 * D + 3, D),
        "tt": (False, False, 2 * D, 2 * D + 2, 0),
    }
    order = ["dd", "td", "db", "tt"]

    rels1, args1, rels2 = [], [], []
    for t in order:
        sd, dd_, alsr, aldr, hoff = meta[t]
        packed, e_real, e_pad = ei[t]
        rels1.append((sd, dd_, alsr, aldr, e_real, e_pad))
        args1 += [packed]
        rels2.append((sd, hoff, e_pad))
    exs, sparts = _run_sc1(rels1, HD, HT, args1)

    args2 = []
    for i, t in enumerate(order):
        packed, _, _ = ei[t]
        args2 += [packed, exs[i]]
    numers = _run_sc2(rels2, HD, HT, args2)

    nm = dict(zip(order, numers))
    sp = dict(zip(order, sparts))
    outdT, outtT = _finalize(
        (nm["dd"], sp["dd"], b["dd"], nm["td"], sp["td"], b["td"]),
        (nm["db"], sp["db"], b["db"], nm["tt"], sp["tt"], b["tt"]))
    return outdT, outtT


def kernel(x_drug, x_target, ei_dd, ei_db, ei_td, ei_tt,
           W1dd, as1dd, ad1dd, b1dd,
           W1db, as1db, ad1db, b1db,
           W1td, as1td, ad1td, b1td,
           W1tt, as1tt, ad1tt, b1tt,
           W2dd, as2dd, ad2dd, b2dd,
           W2db, as2db, ad2db, b2db,
           W2td, as2td, ad2td, b2td,
           W2tt, as2tt, ad2tt, b2tt):
    ei = {
        "dd": _pad_edges(ei_dd[0], ei_dd[1], True),
        "db": _pad_edges(ei_db[0], ei_db[1], False),
        "td": _pad_edges(ei_td[0], ei_td[1], False),
        "tt": _pad_edges(ei_tt[0], ei_tt[1], True),
    }
    p = dict(locals())
    xdT = x_drug.T
    xtT = x_target.T
    for L in ("1", "2"):
        W = {t: p["W" + L + t] for t in ("dd", "db", "td", "tt")}
        ass = {t: p["as" + L + t] for t in ("dd", "db", "td", "tt")}
        ad = {t: p["ad" + L + t] for t in ("dd", "db", "td", "tt")}
        b = {t: p["b" + L + t] for t in ("dd", "db", "td", "tt")}
        xdT, xtT = _layer(xdT, xtT, W, ass, ad, b, ei)
    return (xdT.T, xtT.T)
